# baseline (speedup 1.0000x reference)
import functools

import jax
import jax.numpy as jnp
from jax import lax
from jax.experimental import pallas as pl
from jax.experimental.pallas import tpu as pltpu
from jax.experimental.pallas import tpu_sc as plsc

NUM_SAMPLES = 1000000
BATCH = 16384
DIHEDRAL_EPS = 0.076
GNN_EPS = 6.38
FOLDSEEK_EPS = 3.0

_INFO = plsc.get_sparse_core_info()
_NC = _INFO.num_cores
_NS = _INFO.num_subcores
_NW = _NC * _NS
_L = _INFO.num_lanes
_BPW = BATCH // _NW
_HALF = _BPW // 2


def _body(idx_hbm, dl_hbm, gl_hbm, fl_hbm, lamd_hbm, lamg_hbm, lamf_hbm,
          out_hbm, idx_v, ld_v, lg_v, lf_v, dl_v, gl_v, fl_v, acc_v,
          sem, sem_b):
    wid = lax.axis_index("s") * _NC + lax.axis_index("c")
    base = wid * _BPW
    copies = [
        pltpu.async_copy(dl_hbm.at[pl.ds(base, _BPW)], dl_v, sem),
        pltpu.async_copy(gl_hbm.at[pl.ds(base, _BPW)], gl_v, sem),
        pltpu.async_copy(fl_hbm.at[pl.ds(base, _BPW)], fl_v, sem),
    ]
    lo = pl.ds(0, _HALF)
    hi = pl.ds(_HALF, _HALF)
    pltpu.sync_copy(idx_hbm.at[pl.ds(base, _HALF)], idx_v.at[lo])
    copies += [
        pltpu.async_copy(lamd_hbm.at[idx_v.at[lo]], ld_v.at[lo], sem),
        pltpu.async_copy(lamg_hbm.at[idx_v.at[lo]], lg_v.at[lo], sem),
        pltpu.async_copy(lamf_hbm.at[idx_v.at[lo]], lf_v.at[lo], sem),
    ]
    pltpu.sync_copy(idx_hbm.at[pl.ds(base + _HALF, _HALF)], idx_v.at[hi])
    copies_b = [
        pltpu.async_copy(lamd_hbm.at[idx_v.at[hi]], ld_v.at[hi], sem_b),
        pltpu.async_copy(lamg_hbm.at[idx_v.at[hi]], lg_v.at[hi], sem_b),
        pltpu.async_copy(lamf_hbm.at[idx_v.at[hi]], lf_v.at[hi], sem_b),
    ]
    for c in copies:
        c.wait()

    def term(s):
        return (ld_v[s] * (dl_v[s] - DIHEDRAL_EPS)
                + lg_v[s] * (gl_v[s] - GNN_EPS)
                + lf_v[s] * (fl_v[s] - FOLDSEEK_EPS))

    def step(i, acc):
        b = pl.multiple_of(i * 2 * _L, _L)
        return acc + term(pl.ds(b, _L)) + term(pl.ds(b + _L, _L))

    nstep = _HALF // (2 * _L)
    acc = lax.fori_loop(0, nstep, step, jnp.zeros((_L,), jnp.float32))
    for c in copies_b:
        c.wait()
    acc_v[...] = lax.fori_loop(nstep, 2 * nstep, step, acc)
    pltpu.sync_copy(acc_v, out_hbm.at[wid])


_sc_call = functools.partial(
    pl.kernel,
    mesh=plsc.VectorSubcoreMesh(core_axis_name="c", subcore_axis_name="s"),
    out_type=jax.ShapeDtypeStruct((_NW, _L), jnp.float32),
    scratch_types=[
        pltpu.VMEM((_BPW,), jnp.int32),
        pltpu.VMEM((_BPW,), jnp.float32),
        pltpu.VMEM((_BPW,), jnp.float32),
        pltpu.VMEM((_BPW,), jnp.float32),
        pltpu.VMEM((_BPW,), jnp.float32),
        pltpu.VMEM((_BPW,), jnp.float32),
        pltpu.VMEM((_BPW,), jnp.float32),
        pltpu.VMEM((_L,), jnp.float32),
        pltpu.SemaphoreType.DMA,
        pltpu.SemaphoreType.DMA,
    ],
)(_body)


def kernel(primary_loss, dihedral_losses, gnn_losses, foldseek_losses,
           indices, lam_dihedral, lam_gnn, lam_foldseek):
    idx = indices.astype(jnp.int32)
    partials = _sc_call(idx, dihedral_losses, gnn_losses, foldseek_losses,
                        lam_dihedral, lam_gnn, lam_foldseek)
    return primary_loss + jnp.sum(partials)


# speedup vs baseline: 1.0163x; 1.0163x over previous
import functools

import jax
import jax.numpy as jnp
from jax import lax
from jax.experimental import pallas as pl
from jax.experimental.pallas import tpu as pltpu
from jax.experimental.pallas import tpu_sc as plsc

NUM_SAMPLES = 1000000
BATCH = 16384
DIHEDRAL_EPS = 0.076
GNN_EPS = 6.38
FOLDSEEK_EPS = 3.0

_INFO = plsc.get_sparse_core_info()
_NC = _INFO.num_cores
_NS = _INFO.num_subcores
_NW = _NC * _NS
_L = _INFO.num_lanes
_BPW = BATCH // _NW
_HALF = _BPW // 2


def _body(idx_hbm, dl_hbm, gl_hbm, fl_hbm, lamd_hbm, lamg_hbm, lamf_hbm,
          out_hbm, idx_v, ld_v, lg_v, lf_v, dl_v, gl_v, fl_v, acc_v,
          sem, sem_b):
    wid = lax.axis_index("s") * _NC + lax.axis_index("c")
    base = wid * _BPW
    copies = [
        pltpu.async_copy(dl_hbm.at[pl.ds(base, _BPW)], dl_v, sem),
        pltpu.async_copy(gl_hbm.at[pl.ds(base, _BPW)], gl_v, sem),
        pltpu.async_copy(fl_hbm.at[pl.ds(base, _BPW)], fl_v, sem),
    ]
    lo = pl.ds(0, _HALF)
    hi = pl.ds(_HALF, _HALF)
    pltpu.sync_copy(idx_hbm.at[pl.ds(base, _BPW)], idx_v)
    copies += [
        pltpu.async_copy(lamd_hbm.at[idx_v.at[lo]], ld_v.at[lo], sem),
        pltpu.async_copy(lamg_hbm.at[idx_v.at[lo]], lg_v.at[lo], sem),
        pltpu.async_copy(lamf_hbm.at[idx_v.at[lo]], lf_v.at[lo], sem),
    ]
    copies_b = [
        pltpu.async_copy(lamd_hbm.at[idx_v.at[hi]], ld_v.at[hi], sem_b),
        pltpu.async_copy(lamg_hbm.at[idx_v.at[hi]], lg_v.at[hi], sem_b),
        pltpu.async_copy(lamf_hbm.at[idx_v.at[hi]], lf_v.at[hi], sem_b),
    ]
    for c in copies:
        c.wait()

    def term(s):
        return (ld_v[s] * (dl_v[s] - DIHEDRAL_EPS)
                + lg_v[s] * (gl_v[s] - GNN_EPS)
                + lf_v[s] * (fl_v[s] - FOLDSEEK_EPS))

    def step(i, acc):
        return acc + term(pl.ds(pl.multiple_of(i * _L, _L), _L))

    nstep = _HALF // _L
    acc = lax.fori_loop(0, nstep, step, jnp.zeros((_L,), jnp.float32))
    for c in copies_b:
        c.wait()
    acc_v[...] = lax.fori_loop(nstep, 2 * nstep, step, acc)
    pltpu.sync_copy(acc_v, out_hbm.at[wid])


_sc_call = functools.partial(
    pl.kernel,
    mesh=plsc.VectorSubcoreMesh(core_axis_name="c", subcore_axis_name="s"),
    out_type=jax.ShapeDtypeStruct((_NW, _L), jnp.float32),
    scratch_types=[
        pltpu.VMEM((_BPW,), jnp.int32),
        pltpu.VMEM((_BPW,), jnp.float32),
        pltpu.VMEM((_BPW,), jnp.float32),
        pltpu.VMEM((_BPW,), jnp.float32),
        pltpu.VMEM((_BPW,), jnp.float32),
        pltpu.VMEM((_BPW,), jnp.float32),
        pltpu.VMEM((_BPW,), jnp.float32),
        pltpu.VMEM((_L,), jnp.float32),
        pltpu.SemaphoreType.DMA,
        pltpu.SemaphoreType.DMA,
    ],
)(_body)


def kernel(primary_loss, dihedral_losses, gnn_losses, foldseek_losses,
           indices, lam_dihedral, lam_gnn, lam_foldseek):
    idx = indices.astype(jnp.int32)
    partials = _sc_call(idx, dihedral_losses, gnn_losses, foldseek_losses,
                        lam_dihedral, lam_gnn, lam_foldseek)
    return primary_loss + jnp.sum(partials)
